# baseline (speedup 1.0000x reference)
import functools

import jax
import jax.numpy as jnp
from jax import lax
from jax.experimental import pallas as pl
from jax.experimental.pallas import tpu as pltpu
from jax.experimental.pallas import tpu_sc as plsc

F32 = jnp.float32

NC = 2
NS = 16
NW = NC * NS
CHK = 512

BM = 400

_SC_PARAMS = pltpu.CompilerParams(use_tc_tiling_on_sc=False)


def _spmm_body(n_steps, C, tab_hbm, sidx_hbm, didx_hbm, zeros_hbm, out_hbm,
               sidx_v, didx_v, rows0_v, rows1_v, acc, gsem, ssem):
  c = lax.axis_index("c")
  s = lax.axis_index("s")
  wid = s * NC + c
  nch = sidx_hbm.shape[2]
  rows_bufs = [rows0_v, rows1_v]
  NB = len(rows_bufs)
  rows_pt = acc.shape[0] // NS
  base = s * rows_pt

  pltpu.sync_copy(didx_hbm.at[wid], didx_v)

  def gather(ch, j):
    pltpu.async_copy(tab_hbm.at[sidx_v.at[ch]], rows_bufs[j], gsem)

  def wait_gather(j):
    pltpu.make_async_copy(
        tab_hbm.at[sidx_v.at[0]], rows_bufs[j], gsem).wait()

  def scatter(ch, j):
    pltpu.async_copy(rows_bufs[j], acc.at[didx_v.at[ch]], ssem, add=True)

  def wait_scatter(j):
    pltpu.make_async_copy(
        rows_bufs[j], acc.at[didx_v.at[0]], ssem).wait()

  for k in range(n_steps):
    pltpu.sync_copy(zeros_hbm, acc.at[pl.ds(base, rows_pt)])
    pltpu.sync_copy(sidx_hbm.at[k, wid], sidx_v)
    plsc.subcore_barrier()

    for j in range(NB):
      gather(j, j)

    def chunkgrp(i, carry):
      for j in range(NB):
        wait_gather(j)
        scatter(NB * i + j, j)
      for j in range(NB):
        wait_scatter(j)
        gather(lax.rem(NB * (i + 1) + j, nch), j)
      return carry
    lax.fori_loop(0, nch // NB - 1, chunkgrp, 0)
    for j in range(NB):
      wait_gather(j)
      scatter(nch - NB + j, j)
    for j in range(NB):
      wait_scatter(j)
    plsc.subcore_barrier()

    pltpu.sync_copy(acc.at[pl.ds(base, rows_pt)],
                    out_hbm.at[k * NC + c, pl.ds(base, rows_pt)])


def _spmm(tab, sidx, didx, acc_rows):
  n_steps = sidx.shape[0]
  nch = sidx.shape[2]
  C = tab.shape[1]
  mesh = plsc.VectorSubcoreMesh(
      core_axis_name="c", subcore_axis_name="s", num_cores=NC,
      num_subcores=NS)
  zeros = jnp.zeros((acc_rows // NS, C), F32)
  return pl.kernel(
      functools.partial(_spmm_body, n_steps, C),
      out_type=jax.ShapeDtypeStruct((n_steps * NC, acc_rows, C), F32),
      mesh=mesh,
      compiler_params=_SC_PARAMS,
      scratch_types=[
          pltpu.VMEM((nch, CHK), jnp.int32),
          pltpu.VMEM((nch, CHK), jnp.int32),
          pltpu.VMEM((CHK, C), F32),
          pltpu.VMEM((CHK, C), F32),
          pltpu.VMEM_SHARED((acc_rows, C), F32),
          pltpu.SemaphoreType.DMA,
          pltpu.SemaphoreType.DMA,
      ],
  )(tab, sidx, didx, zeros)


def _deg_body(C, ones_hbm, didx_hbm, out_hbm, didx_v, ones_v, zbuf_v, acc):
  c = lax.axis_index("c")
  s = lax.axis_index("s")
  wid = s * NC + c
  nch = didx_hbm.shape[1]
  rows_pt = acc.shape[0] // NS
  stage = rows_pt // 2
  base = s * rows_pt

  def zrow(i, carry):
    for j in range(C // 16):
      zbuf_v[i, pl.ds(j * 16, 16)] = jnp.zeros((16,), F32)
    return carry
  lax.fori_loop(0, stage, zrow, 0)

  pltpu.sync_copy(zbuf_v, acc.at[pl.ds(base, stage)])
  pltpu.sync_copy(zbuf_v, acc.at[pl.ds(base + stage, stage)])
  pltpu.sync_copy(ones_hbm, ones_v)
  pltpu.sync_copy(didx_hbm.at[wid], didx_v)
  plsc.subcore_barrier()

  def chunk(ch, carry):
    pltpu.sync_copy(ones_v, acc.at[didx_v.at[ch]], add=True)
    return carry
  lax.fori_loop(0, nch, chunk, 0)
  plsc.subcore_barrier()

  pltpu.sync_copy(acc.at[pl.ds(base, stage)], zbuf_v)
  pltpu.sync_copy(zbuf_v, out_hbm.at[c, pl.ds(base, stage)])
  pltpu.sync_copy(acc.at[pl.ds(base + stage, stage)], zbuf_v)
  pltpu.sync_copy(zbuf_v, out_hbm.at[c, pl.ds(base + stage, stage)])


def _deg(didx, acc_rows, C):
  nch = didx.shape[1]
  mesh = plsc.VectorSubcoreMesh(
      core_axis_name="c", subcore_axis_name="s", num_cores=NC,
      num_subcores=NS)
  ones = jnp.ones((CHK, C), F32)
  return pl.kernel(
      functools.partial(_deg_body, C),
      out_type=jax.ShapeDtypeStruct((NC, acc_rows, C), F32),
      mesh=mesh,
      compiler_params=_SC_PARAMS,
      scratch_types=[
          pltpu.VMEM((nch, CHK), jnp.int32),
          pltpu.VMEM((CHK, C), F32),
          pltpu.VMEM((acc_rows // NS // 2, C), F32),
          pltpu.VMEM_SHARED((acc_rows, C), F32),
      ],
  )(ones, didx)


def _dinv_body(dega_ref, degb_ref, out_ref):
  deg = dega_ref[0][:, 0:1] + degb_ref[0][:, 0:1]
  dinv = jnp.where(deg > 0, lax.rsqrt(deg), 0.0)
  out_ref[...] = jnp.broadcast_to(dinv, out_ref.shape)


def _xside_body(H, x_ref, w0_ref, dinv_ref, xw0_ref, xlo_ref, xhi_ref):
  x = x_ref[...]
  xw0_ref[...] = jnp.dot(x, w0_ref[...], preferred_element_type=F32)
  xs = x * dinv_ref[...]
  xlo_ref[...] = xs[:, :H]
  xhi_ref[...] = xs[:, H:]


def _sx_body(lla_ref, llb_ref, lha_ref, lhb_ref, dinv_ref, w1_ref, out_ref):
  lx = jnp.concatenate(
      [lla_ref[0] + llb_ref[0], lha_ref[0] + lhb_ref[0]], axis=1)
  lx = lx * dinv_ref[...]
  out_ref[...] = jnp.dot(lx, w1_ref[...], preferred_element_type=F32)


def _gate_body(H, sx_ref, xw0_ref, lha_ref, lhb_ref, dinv_ref, h_ref,
               bz_ref, br_ref, wh0zr_ref, whz1_ref, whr1_ref, w0_ref, w1_ref,
               z_ref, cw0_ref, uc1_ref):
  dinv = dinv_ref[...][:, :H]
  lh = (lha_ref[0] + lhb_ref[0]) * dinv
  shz = jnp.dot(lh, whz1_ref[...], preferred_element_type=F32)
  shr = jnp.dot(lh, whr1_ref[...], preferred_element_type=F32)
  h = h_ref[...]
  hw0 = jnp.dot(h, wh0zr_ref[...], preferred_element_type=F32)
  xw0 = xw0_ref[...]
  sx = sx_ref[...]
  z = jax.nn.sigmoid(
      xw0[:, :H] - sx[:, :H] + hw0[:, :H] - shz + bz_ref[...])
  r = jax.nn.sigmoid(
      xw0[:, H:2 * H] - sx[:, H:2 * H] + hw0[:, H:] - shr + br_ref[...])
  c = h * r
  z_ref[...] = z
  cw0_ref[...] = jnp.dot(c, w0_ref[...], preferred_element_type=F32)
  cs = c * dinv
  uc1_ref[...] = jnp.dot(cs, w1_ref[...], preferred_element_type=F32)


def _update_body(H, sx_ref, xw0_ref, cw0_ref, sca_ref, scb_ref,
                 dinv_ref, z_ref, h_ref, bh_ref, out_ref, hs_ref):
  dinv = dinv_ref[...][:, :H]
  sc = (sca_ref[0] + scb_ref[0]) * dinv
  ht = jnp.tanh(xw0_ref[...][:, 2 * H:] - sx_ref[...][:, 2 * H:]
                + cw0_ref[...] - sc + bh_ref[...])
  z = z_ref[...]
  hn = z * h_ref[...] + (1.0 - z) * ht
  out_ref[...] = hn
  hs_ref[...] = hn * dinv


def _step0_body(H, sx_ref, xw0_ref, dinv_ref, bz_ref, bh_ref, out_ref,
                hs_ref):
  z = jax.nn.sigmoid(xw0_ref[...][:, :H] - sx_ref[...][:, :H] + bz_ref[...])
  ht = jnp.tanh(xw0_ref[...][:, 2 * H:] - sx_ref[...][:, 2 * H:]
                + bh_ref[...])
  hn = (1.0 - z) * ht
  out_ref[...] = hn
  hs_ref[...] = hn * dinv_ref[...][:, :H]


def _final_body(h_ref, wl_ref, bl_ref, out_ref):
  h = jnp.maximum(h_ref[...], 0.0)
  out_ref[...] = (jnp.dot(h, wl_ref[...], preferred_element_type=F32)
                  + bl_ref[...])


def kernel(x_seq, Wxz, bxz, Whz, bhz, Wxr, bxr, Whr, bhr, Wxh, bxh, Whh, bhh,
           Wl, bl, edge_index):
  if x_seq.ndim == 3:
    x_seq = x_seq[None]
  B, T, n, F = x_seq.shape
  H = Whz.shape[2]
  OUT = Wl.shape[1]
  BT = B * T
  nblk = n // BM
  acc_rows = ((n + 1 + 16 * NS - 1) // (16 * NS)) * (16 * NS)

  E = edge_index.shape[1]
  nch = -(-E // (NW * CHK))
  nch += (-nch) % 2
  Epad = NW * CHK * nch
  src = edge_index[0].astype(jnp.int32)
  dst = edge_index[1].astype(jnp.int32)
  pad0 = jnp.zeros((Epad - E,), jnp.int32)
  padn = jnp.full((Epad - E,), n, jnp.int32)
  sidx = jnp.concatenate([src, pad0]).reshape(NW, nch, CHK)
  didx = jnp.concatenate([dst, padn]).reshape(NW, nch, CHK)
  sdeg = jnp.concatenate([src, padn]).reshape(NW, nch, CHK)
  offs_bt = (jnp.arange(BT, dtype=jnp.int32) * n)[:, None, None, None]
  sidx_bt = sidx[None] + offs_bt
  sidx_1 = sidx[None]

  deg2 = _deg(sdeg, acc_rows, 16)
  dinvmat = pl.pallas_call(
      _dinv_body,
      grid=(nblk,),
      in_specs=[
          pl.BlockSpec((1, BM, 16), lambda i: (0, i, 0)),
          pl.BlockSpec((1, BM, 16), lambda i: (1, i, 0)),
      ],
      out_specs=pl.BlockSpec((BM, 128), lambda i: (i, 0)),
      out_shape=jax.ShapeDtypeStruct((n, 128), F32),
  )(deg2, deg2)

  X2 = x_seq.reshape(BT * n, F)
  W0cat = jnp.concatenate([Wxz[0], Wxr[0], Wxh[0]], axis=1)
  W1cat = jnp.concatenate([Wxz[1], Wxr[1], Wxh[1]], axis=1)
  XW0, XsLo, XsHi = pl.pallas_call(
      functools.partial(_xside_body, H),
      grid=(BT * nblk,),
      in_specs=[
          pl.BlockSpec((BM, F), lambda i: (i, 0)),
          pl.BlockSpec((F, 3 * H), lambda i: (0, 0)),
          pl.BlockSpec((BM, 128), lambda i: (i % nblk, 0)),
      ],
      out_specs=[
          pl.BlockSpec((BM, 3 * H), lambda i: (i, 0)),
          pl.BlockSpec((BM, H), lambda i: (i, 0)),
          pl.BlockSpec((BM, H), lambda i: (i, 0)),
      ],
      out_shape=[
          jax.ShapeDtypeStruct((BT * n, 3 * H), F32),
          jax.ShapeDtypeStruct((BT * n, H), F32),
          jax.ShapeDtypeStruct((BT * n, H), F32),
      ],
  )(X2, W0cat, dinvmat)

  LxLo = _spmm(XsLo, sidx_bt, didx, acc_rows)
  LxHi = _spmm(XsHi, sidx_bt, didx, acc_rows)

  SX = pl.pallas_call(
      _sx_body,
      grid=(BT * nblk,),
      in_specs=[
          pl.BlockSpec((1, BM, H), lambda i: (2 * (i // nblk), i % nblk, 0)),
          pl.BlockSpec((1, BM, H),
                       lambda i: (2 * (i // nblk) + 1, i % nblk, 0)),
          pl.BlockSpec((1, BM, H), lambda i: (2 * (i // nblk), i % nblk, 0)),
          pl.BlockSpec((1, BM, H),
                       lambda i: (2 * (i // nblk) + 1, i % nblk, 0)),
          pl.BlockSpec((BM, 128), lambda i: (i % nblk, 0)),
          pl.BlockSpec((F, 3 * H), lambda i: (0, 0)),
      ],
      out_specs=pl.BlockSpec((BM, 3 * H), lambda i: (i, 0)),
      out_shape=jax.ShapeDtypeStruct((BT * n, 3 * H), F32),
  )(LxLo, LxLo, LxHi, LxHi, dinvmat, W1cat)

  Wh0zr = jnp.concatenate([Whz[0], Whr[0]], axis=1)
  bz2 = (bxz + bhz).reshape(1, H)
  br2 = (bxr + bhr).reshape(1, H)
  bh2 = (bxh + bhh).reshape(1, H)
  bl2 = bl.reshape(1, OUT)

  outs = []
  for b in range(B):
    bt0 = b * T
    h, hs = pl.pallas_call(
        functools.partial(_step0_body, H),
        grid=(nblk,),
        in_specs=[
            pl.BlockSpec((BM, 3 * H), lambda i, bt=bt0: (bt * nblk + i, 0)),
            pl.BlockSpec((BM, 3 * H), lambda i, bt=bt0: (bt * nblk + i, 0)),
            pl.BlockSpec((BM, 128), lambda i: (i, 0)),
            pl.BlockSpec((1, H), lambda i: (0, 0)),
            pl.BlockSpec((1, H), lambda i: (0, 0)),
        ],
        out_specs=[
            pl.BlockSpec((BM, H), lambda i: (i, 0)),
            pl.BlockSpec((BM, H), lambda i: (i, 0)),
        ],
        out_shape=[
            jax.ShapeDtypeStruct((n, H), F32),
            jax.ShapeDtypeStruct((n, H), F32),
        ],
    )(SX, XW0, dinvmat, bz2, bh2)
    for t in range(1, T):
      bt = b * T + t
      Lh = _spmm(hs, sidx_1, didx, acc_rows)

      Z, CW0, UC1 = pl.pallas_call(
          functools.partial(_gate_body, H),
          grid=(nblk,),
          in_specs=[
              pl.BlockSpec((BM, 3 * H), lambda i, bt=bt: (bt * nblk + i, 0)),
              pl.BlockSpec((BM, 3 * H), lambda i, bt=bt: (bt * nblk + i, 0)),
              pl.BlockSpec((1, BM, H), lambda i: (0, i, 0)),
              pl.BlockSpec((1, BM, H), lambda i: (1, i, 0)),
              pl.BlockSpec((BM, 128), lambda i: (i, 0)),
              pl.BlockSpec((BM, H), lambda i: (i, 0)),
              pl.BlockSpec((1, H), lambda i: (0, 0)),
              pl.BlockSpec((1, H), lambda i: (0, 0)),
              pl.BlockSpec((H, 2 * H), lambda i: (0, 0)),
              pl.BlockSpec((H, H), lambda i: (0, 0)),
              pl.BlockSpec((H, H), lambda i: (0, 0)),
              pl.BlockSpec((H, H), lambda i: (0, 0)),
              pl.BlockSpec((H, H), lambda i: (0, 0)),
          ],
          out_specs=[
              pl.BlockSpec((BM, H), lambda i: (i, 0)),
              pl.BlockSpec((BM, H), lambda i: (i, 0)),
              pl.BlockSpec((BM, H), lambda i: (i, 0)),
          ],
          out_shape=[
              jax.ShapeDtypeStruct((n, H), F32),
              jax.ShapeDtypeStruct((n, H), F32),
              jax.ShapeDtypeStruct((n, H), F32),
          ],
      )(SX, XW0, Lh, Lh, dinvmat, h, bz2, br2, Wh0zr, Whz[1], Whr[1],
        Whh[0], Whh[1])

      SCt = _spmm(UC1, sidx_1, didx, acc_rows)

      h, hs = pl.pallas_call(
          functools.partial(_update_body, H),
          grid=(nblk,),
          in_specs=[
              pl.BlockSpec((BM, 3 * H), lambda i, bt=bt: (bt * nblk + i, 0)),
              pl.BlockSpec((BM, 3 * H), lambda i, bt=bt: (bt * nblk + i, 0)),
              pl.BlockSpec((BM, H), lambda i: (i, 0)),
              pl.BlockSpec((1, BM, H), lambda i: (0, i, 0)),
              pl.BlockSpec((1, BM, H), lambda i: (1, i, 0)),
              pl.BlockSpec((BM, 128), lambda i: (i, 0)),
              pl.BlockSpec((BM, H), lambda i: (i, 0)),
              pl.BlockSpec((BM, H), lambda i: (i, 0)),
              pl.BlockSpec((1, H), lambda i: (0, 0)),
          ],
          out_specs=[
              pl.BlockSpec((BM, H), lambda i: (i, 0)),
              pl.BlockSpec((BM, H), lambda i: (i, 0)),
          ],
          out_shape=[
              jax.ShapeDtypeStruct((n, H), F32),
              jax.ShapeDtypeStruct((n, H), F32),
          ],
      )(SX, XW0, CW0, SCt, SCt, dinvmat, Z, h, bh2)

    out_b = pl.pallas_call(
        _final_body,
        grid=(nblk,),
        in_specs=[
            pl.BlockSpec((BM, H), lambda i: (i, 0)),
            pl.BlockSpec((H, OUT), lambda i: (0, 0)),
            pl.BlockSpec((1, OUT), lambda i: (0, 0)),
        ],
        out_specs=pl.BlockSpec((BM, OUT), lambda i: (i, 0)),
        out_shape=jax.ShapeDtypeStruct((n, OUT), F32),
    )(h, Wl, bl2)
    outs.append(out_b)

  return jnp.stack(outs, axis=0)


# speedup vs baseline: 1.0306x; 1.0306x over previous
import functools

import jax
import jax.numpy as jnp
from jax import lax
from jax.experimental import pallas as pl
from jax.experimental.pallas import tpu as pltpu
from jax.experimental.pallas import tpu_sc as plsc

F32 = jnp.float32

NC = 2
NS = 16
NW = NC * NS
CHK = 256

BM = 400

_SC_PARAMS = pltpu.CompilerParams(use_tc_tiling_on_sc=False)


def _spmm_body(n_steps, C, tab_hbm, sidx_hbm, didx_hbm, zeros_hbm, out_hbm,
               sidx_v, didx_v, rows0_v, rows1_v, acc, gsem, ssem):
  c = lax.axis_index("c")
  s = lax.axis_index("s")
  wid = s * NC + c
  nch = sidx_hbm.shape[2]
  rows_bufs = [rows0_v, rows1_v]
  NB = len(rows_bufs)
  rows_pt = acc.shape[0] // NS
  base = s * rows_pt

  pltpu.sync_copy(didx_hbm.at[wid], didx_v)

  def gather(ch, j):
    pltpu.async_copy(tab_hbm.at[sidx_v.at[ch]], rows_bufs[j], gsem)

  def wait_gather(j):
    pltpu.make_async_copy(
        tab_hbm.at[sidx_v.at[0]], rows_bufs[j], gsem).wait()

  def scatter(ch, j):
    pltpu.async_copy(rows_bufs[j], acc.at[didx_v.at[ch]], ssem, add=True)

  def wait_scatter(j):
    pltpu.make_async_copy(
        rows_bufs[j], acc.at[didx_v.at[0]], ssem).wait()

  for k in range(n_steps):
    pltpu.sync_copy(zeros_hbm, acc.at[pl.ds(base, rows_pt)])
    pltpu.sync_copy(sidx_hbm.at[k, wid], sidx_v)
    plsc.subcore_barrier()

    for j in range(NB):
      gather(j, j)

    def chunkgrp(i, carry):
      for j in range(NB):
        wait_gather(j)
        scatter(NB * i + j, j)
      for j in range(NB):
        wait_scatter(j)
        gather(lax.rem(NB * (i + 1) + j, nch), j)
      return carry
    lax.fori_loop(0, nch // NB - 1, chunkgrp, 0)
    for j in range(NB):
      wait_gather(j)
      scatter(nch - NB + j, j)
    for j in range(NB):
      wait_scatter(j)
    plsc.subcore_barrier()

    pltpu.sync_copy(acc.at[pl.ds(base, rows_pt)],
                    out_hbm.at[k * NC + c, pl.ds(base, rows_pt)])


def _spmm(tab, sidx, didx, acc_rows):
  n_steps = sidx.shape[0]
  nch = sidx.shape[2]
  C = tab.shape[1]
  mesh = plsc.VectorSubcoreMesh(
      core_axis_name="c", subcore_axis_name="s", num_cores=NC,
      num_subcores=NS)
  zeros = jnp.zeros((acc_rows // NS, C), F32)
  return pl.kernel(
      functools.partial(_spmm_body, n_steps, C),
      out_type=jax.ShapeDtypeStruct((n_steps * NC, acc_rows, C), F32),
      mesh=mesh,
      compiler_params=_SC_PARAMS,
      scratch_types=[
          pltpu.VMEM((nch, CHK), jnp.int32),
          pltpu.VMEM((nch, CHK), jnp.int32),
          pltpu.VMEM((CHK, C), F32),
          pltpu.VMEM((CHK, C), F32),
          pltpu.VMEM_SHARED((acc_rows, C), F32),
          pltpu.SemaphoreType.DMA,
          pltpu.SemaphoreType.DMA,
      ],
  )(tab, sidx, didx, zeros)


def _deg_body(C, ones_hbm, didx_hbm, out_hbm, didx_v, ones_v, zbuf_v, acc):
  c = lax.axis_index("c")
  s = lax.axis_index("s")
  wid = s * NC + c
  nch = didx_hbm.shape[1]
  rows_pt = acc.shape[0] // NS
  stage = rows_pt // 2
  base = s * rows_pt

  def zrow(i, carry):
    for j in range(C // 16):
      zbuf_v[i, pl.ds(j * 16, 16)] = jnp.zeros((16,), F32)
    return carry
  lax.fori_loop(0, stage, zrow, 0)

  pltpu.sync_copy(zbuf_v, acc.at[pl.ds(base, stage)])
  pltpu.sync_copy(zbuf_v, acc.at[pl.ds(base + stage, stage)])
  pltpu.sync_copy(ones_hbm, ones_v)
  pltpu.sync_copy(didx_hbm.at[wid], didx_v)
  plsc.subcore_barrier()

  def chunk(ch, carry):
    pltpu.sync_copy(ones_v, acc.at[didx_v.at[ch]], add=True)
    return carry
  lax.fori_loop(0, nch, chunk, 0)
  plsc.subcore_barrier()

  pltpu.sync_copy(acc.at[pl.ds(base, stage)], zbuf_v)
  pltpu.sync_copy(zbuf_v, out_hbm.at[c, pl.ds(base, stage)])
  pltpu.sync_copy(acc.at[pl.ds(base + stage, stage)], zbuf_v)
  pltpu.sync_copy(zbuf_v, out_hbm.at[c, pl.ds(base + stage, stage)])


def _deg(didx, acc_rows, C):
  nch = didx.shape[1]
  mesh = plsc.VectorSubcoreMesh(
      core_axis_name="c", subcore_axis_name="s", num_cores=NC,
      num_subcores=NS)
  ones = jnp.ones((CHK, C), F32)
  return pl.kernel(
      functools.partial(_deg_body, C),
      out_type=jax.ShapeDtypeStruct((NC, acc_rows, C), F32),
      mesh=mesh,
      compiler_params=_SC_PARAMS,
      scratch_types=[
          pltpu.VMEM((nch, CHK), jnp.int32),
          pltpu.VMEM((CHK, C), F32),
          pltpu.VMEM((acc_rows // NS // 2, C), F32),
          pltpu.VMEM_SHARED((acc_rows, C), F32),
      ],
  )(ones, didx)


def _dinv_body(dega_ref, degb_ref, out_ref):
  deg = dega_ref[0][:, 0:1] + degb_ref[0][:, 0:1]
  dinv = jnp.where(deg > 0, lax.rsqrt(deg), 0.0)
  out_ref[...] = jnp.broadcast_to(dinv, out_ref.shape)


def _xside_body(H, x_ref, w0_ref, dinv_ref, xw0_ref, xlo_ref, xhi_ref):
  x = x_ref[...]
  xw0_ref[...] = jnp.dot(x, w0_ref[...], preferred_element_type=F32)
  xs = x * dinv_ref[...]
  xlo_ref[...] = xs[:, :H]
  xhi_ref[...] = xs[:, H:]


def _sx_body(lla_ref, llb_ref, lha_ref, lhb_ref, dinv_ref, w1_ref, out_ref):
  lx = jnp.concatenate(
      [lla_ref[0] + llb_ref[0], lha_ref[0] + lhb_ref[0]], axis=1)
  lx = lx * dinv_ref[...]
  out_ref[...] = jnp.dot(lx, w1_ref[...], preferred_element_type=F32)


def _gate_body(H, sx_ref, xw0_ref, lha_ref, lhb_ref, dinv_ref, h_ref,
               bz_ref, br_ref, wh0zr_ref, whz1_ref, whr1_ref, w0_ref, w1_ref,
               z_ref, cw0_ref, uc1_ref):
  dinv = dinv_ref[...][:, :H]
  lh = (lha_ref[0] + lhb_ref[0]) * dinv
  shz = jnp.dot(lh, whz1_ref[...], preferred_element_type=F32)
  shr = jnp.dot(lh, whr1_ref[...], preferred_element_type=F32)
  h = h_ref[...]
  hw0 = jnp.dot(h, wh0zr_ref[...], preferred_element_type=F32)
  xw0 = xw0_ref[...]
  sx = sx_ref[...]
  z = jax.nn.sigmoid(
      xw0[:, :H] - sx[:, :H] + hw0[:, :H] - shz + bz_ref[...])
  r = jax.nn.sigmoid(
      xw0[:, H:2 * H] - sx[:, H:2 * H] + hw0[:, H:] - shr + br_ref[...])
  c = h * r
  z_ref[...] = z
  cw0_ref[...] = jnp.dot(c, w0_ref[...], preferred_element_type=F32)
  cs = c * dinv
  uc1_ref[...] = jnp.dot(cs, w1_ref[...], preferred_element_type=F32)


def _update_body(H, sx_ref, xw0_ref, cw0_ref, sca_ref, scb_ref,
                 dinv_ref, z_ref, h_ref, bh_ref, out_ref, hs_ref):
  dinv = dinv_ref[...][:, :H]
  sc = (sca_ref[0] + scb_ref[0]) * dinv
  ht = jnp.tanh(xw0_ref[...][:, 2 * H:] - sx_ref[...][:, 2 * H:]
                + cw0_ref[...] - sc + bh_ref[...])
  z = z_ref[...]
  hn = z * h_ref[...] + (1.0 - z) * ht
  out_ref[...] = hn
  hs_ref[...] = hn * dinv


def _step0_body(H, sx_ref, xw0_ref, dinv_ref, bz_ref, bh_ref, out_ref,
                hs_ref):
  z = jax.nn.sigmoid(xw0_ref[...][:, :H] - sx_ref[...][:, :H] + bz_ref[...])
  ht = jnp.tanh(xw0_ref[...][:, 2 * H:] - sx_ref[...][:, 2 * H:]
                + bh_ref[...])
  hn = (1.0 - z) * ht
  out_ref[...] = hn
  hs_ref[...] = hn * dinv_ref[...][:, :H]


def _final_body(h_ref, wl_ref, bl_ref, out_ref):
  h = jnp.maximum(h_ref[...], 0.0)
  out_ref[...] = (jnp.dot(h, wl_ref[...], preferred_element_type=F32)
                  + bl_ref[...])


def kernel(x_seq, Wxz, bxz, Whz, bhz, Wxr, bxr, Whr, bhr, Wxh, bxh, Whh, bhh,
           Wl, bl, edge_index):
  if x_seq.ndim == 3:
    x_seq = x_seq[None]
  B, T, n, F = x_seq.shape
  H = Whz.shape[2]
  OUT = Wl.shape[1]
  BT = B * T
  nblk = n // BM
  acc_rows = ((n + 1 + 16 * NS - 1) // (16 * NS)) * (16 * NS)

  E = edge_index.shape[1]
  nch = -(-E // (NW * CHK))
  nch += (-nch) % 2
  Epad = NW * CHK * nch
  src = edge_index[0].astype(jnp.int32)
  dst = edge_index[1].astype(jnp.int32)
  pad0 = jnp.zeros((Epad - E,), jnp.int32)
  padn = jnp.full((Epad - E,), n, jnp.int32)
  sidx = jnp.concatenate([src, pad0]).reshape(NW, nch, CHK)
  didx = jnp.concatenate([dst, padn]).reshape(NW, nch, CHK)
  sdeg = jnp.concatenate([src, padn]).reshape(NW, nch, CHK)
  offs_bt = (jnp.arange(BT, dtype=jnp.int32) * n)[:, None, None, None]
  sidx_bt = sidx[None] + offs_bt
  sidx_1 = sidx[None]

  deg2 = _deg(sdeg, acc_rows, 16)
  dinvmat = pl.pallas_call(
      _dinv_body,
      grid=(nblk,),
      in_specs=[
          pl.BlockSpec((1, BM, 16), lambda i: (0, i, 0)),
          pl.BlockSpec((1, BM, 16), lambda i: (1, i, 0)),
      ],
      out_specs=pl.BlockSpec((BM, 128), lambda i: (i, 0)),
      out_shape=jax.ShapeDtypeStruct((n, 128), F32),
  )(deg2, deg2)

  X2 = x_seq.reshape(BT * n, F)
  W0cat = jnp.concatenate([Wxz[0], Wxr[0], Wxh[0]], axis=1)
  W1cat = jnp.concatenate([Wxz[1], Wxr[1], Wxh[1]], axis=1)
  XW0, XsLo, XsHi = pl.pallas_call(
      functools.partial(_xside_body, H),
      grid=(BT * nblk,),
      in_specs=[
          pl.BlockSpec((BM, F), lambda i: (i, 0)),
          pl.BlockSpec((F, 3 * H), lambda i: (0, 0)),
          pl.BlockSpec((BM, 128), lambda i: (i % nblk, 0)),
      ],
      out_specs=[
          pl.BlockSpec((BM, 3 * H), lambda i: (i, 0)),
          pl.BlockSpec((BM, H), lambda i: (i, 0)),
          pl.BlockSpec((BM, H), lambda i: (i, 0)),
      ],
      out_shape=[
          jax.ShapeDtypeStruct((BT * n, 3 * H), F32),
          jax.ShapeDtypeStruct((BT * n, H), F32),
          jax.ShapeDtypeStruct((BT * n, H), F32),
      ],
  )(X2, W0cat, dinvmat)

  LxLo = _spmm(XsLo, sidx_bt, didx, acc_rows)
  LxHi = _spmm(XsHi, sidx_bt, didx, acc_rows)

  SX = pl.pallas_call(
      _sx_body,
      grid=(BT * nblk,),
      in_specs=[
          pl.BlockSpec((1, BM, H), lambda i: (2 * (i // nblk), i % nblk, 0)),
          pl.BlockSpec((1, BM, H),
                       lambda i: (2 * (i // nblk) + 1, i % nblk, 0)),
          pl.BlockSpec((1, BM, H), lambda i: (2 * (i // nblk), i % nblk, 0)),
          pl.BlockSpec((1, BM, H),
                       lambda i: (2 * (i // nblk) + 1, i % nblk, 0)),
          pl.BlockSpec((BM, 128), lambda i: (i % nblk, 0)),
          pl.BlockSpec((F, 3 * H), lambda i: (0, 0)),
      ],
      out_specs=pl.BlockSpec((BM, 3 * H), lambda i: (i, 0)),
      out_shape=jax.ShapeDtypeStruct((BT * n, 3 * H), F32),
  )(LxLo, LxLo, LxHi, LxHi, dinvmat, W1cat)

  Wh0zr = jnp.concatenate([Whz[0], Whr[0]], axis=1)
  bz2 = (bxz + bhz).reshape(1, H)
  br2 = (bxr + bhr).reshape(1, H)
  bh2 = (bxh + bhh).reshape(1, H)
  bl2 = bl.reshape(1, OUT)

  outs = []
  for b in range(B):
    bt0 = b * T
    h, hs = pl.pallas_call(
        functools.partial(_step0_body, H),
        grid=(nblk,),
        in_specs=[
            pl.BlockSpec((BM, 3 * H), lambda i, bt=bt0: (bt * nblk + i, 0)),
            pl.BlockSpec((BM, 3 * H), lambda i, bt=bt0: (bt * nblk + i, 0)),
            pl.BlockSpec((BM, 128), lambda i: (i, 0)),
            pl.BlockSpec((1, H), lambda i: (0, 0)),
            pl.BlockSpec((1, H), lambda i: (0, 0)),
        ],
        out_specs=[
            pl.BlockSpec((BM, H), lambda i: (i, 0)),
            pl.BlockSpec((BM, H), lambda i: (i, 0)),
        ],
        out_shape=[
            jax.ShapeDtypeStruct((n, H), F32),
            jax.ShapeDtypeStruct((n, H), F32),
        ],
    )(SX, XW0, dinvmat, bz2, bh2)
    for t in range(1, T):
      bt = b * T + t
      Lh = _spmm(hs, sidx_1, didx, acc_rows)

      Z, CW0, UC1 = pl.pallas_call(
          functools.partial(_gate_body, H),
          grid=(nblk,),
          in_specs=[
              pl.BlockSpec((BM, 3 * H), lambda i, bt=bt: (bt * nblk + i, 0)),
              pl.BlockSpec((BM, 3 * H), lambda i, bt=bt: (bt * nblk + i, 0)),
              pl.BlockSpec((1, BM, H), lambda i: (0, i, 0)),
              pl.BlockSpec((1, BM, H), lambda i: (1, i, 0)),
              pl.BlockSpec((BM, 128), lambda i: (i, 0)),
              pl.BlockSpec((BM, H), lambda i: (i, 0)),
              pl.BlockSpec((1, H), lambda i: (0, 0)),
              pl.BlockSpec((1, H), lambda i: (0, 0)),
              pl.BlockSpec((H, 2 * H), lambda i: (0, 0)),
              pl.BlockSpec((H, H), lambda i: (0, 0)),
              pl.BlockSpec((H, H), lambda i: (0, 0)),
              pl.BlockSpec((H, H), lambda i: (0, 0)),
              pl.BlockSpec((H, H), lambda i: (0, 0)),
          ],
          out_specs=[
              pl.BlockSpec((BM, H), lambda i: (i, 0)),
              pl.BlockSpec((BM, H), lambda i: (i, 0)),
              pl.BlockSpec((BM, H), lambda i: (i, 0)),
          ],
          out_shape=[
              jax.ShapeDtypeStruct((n, H), F32),
              jax.ShapeDtypeStruct((n, H), F32),
              jax.ShapeDtypeStruct((n, H), F32),
          ],
      )(SX, XW0, Lh, Lh, dinvmat, h, bz2, br2, Wh0zr, Whz[1], Whr[1],
        Whh[0], Whh[1])

      SCt = _spmm(UC1, sidx_1, didx, acc_rows)

      h, hs = pl.pallas_call(
          functools.partial(_update_body, H),
          grid=(nblk,),
          in_specs=[
              pl.BlockSpec((BM, 3 * H), lambda i, bt=bt: (bt * nblk + i, 0)),
              pl.BlockSpec((BM, 3 * H), lambda i, bt=bt: (bt * nblk + i, 0)),
              pl.BlockSpec((BM, H), lambda i: (i, 0)),
              pl.BlockSpec((1, BM, H), lambda i: (0, i, 0)),
              pl.BlockSpec((1, BM, H), lambda i: (1, i, 0)),
              pl.BlockSpec((BM, 128), lambda i: (i, 0)),
              pl.BlockSpec((BM, H), lambda i: (i, 0)),
              pl.BlockSpec((BM, H), lambda i: (i, 0)),
              pl.BlockSpec((1, H), lambda i: (0, 0)),
          ],
          out_specs=[
              pl.BlockSpec((BM, H), lambda i: (i, 0)),
              pl.BlockSpec((BM, H), lambda i: (i, 0)),
          ],
          out_shape=[
              jax.ShapeDtypeStruct((n, H), F32),
              jax.ShapeDtypeStruct((n, H), F32),
          ],
      )(SX, XW0, CW0, SCt, SCt, dinvmat, Z, h, bh2)

    out_b = pl.pallas_call(
        _final_body,
        grid=(nblk,),
        in_specs=[
            pl.BlockSpec((BM, H), lambda i: (i, 0)),
            pl.BlockSpec((H, OUT), lambda i: (0, 0)),
            pl.BlockSpec((1, OUT), lambda i: (0, 0)),
        ],
        out_specs=pl.BlockSpec((BM, OUT), lambda i: (i, 0)),
        out_shape=jax.ShapeDtypeStruct((n, OUT), F32),
    )(h, Wl, bl2)
    outs.append(out_b)

  return jnp.stack(outs, axis=0)


# speedup vs baseline: 1.0411x; 1.0101x over previous
import functools

import jax
import jax.numpy as jnp
from jax import lax
from jax.experimental import pallas as pl
from jax.experimental.pallas import tpu as pltpu
from jax.experimental.pallas import tpu_sc as plsc

F32 = jnp.float32

NC = 2
NS = 16
NW = NC * NS
CHK = 256

BM = 400

_SC_PARAMS = pltpu.CompilerParams(use_tc_tiling_on_sc=False)


def _spmm_body(n_steps, C, tab_hbm, sidx_hbm, didx_hbm, zeros_hbm, out_hbm,
               sidx_v, didx_v, rows0_v, rows1_v, rows2_v, rows3_v,
               acc, gsem, ssem):
  c = lax.axis_index("c")
  s = lax.axis_index("s")
  wid = s * NC + c
  nch = sidx_hbm.shape[2]
  rows_bufs = [rows0_v, rows1_v, rows2_v, rows3_v]
  NB = len(rows_bufs)
  rows_pt = acc.shape[0] // NS
  base = s * rows_pt

  pltpu.sync_copy(didx_hbm.at[wid], didx_v)

  def gather(ch, j):
    pltpu.async_copy(tab_hbm.at[sidx_v.at[ch]], rows_bufs[j], gsem)

  def wait_gather(j):
    pltpu.make_async_copy(
        tab_hbm.at[sidx_v.at[0]], rows_bufs[j], gsem).wait()

  def scatter(ch, j):
    pltpu.async_copy(rows_bufs[j], acc.at[didx_v.at[ch]], ssem, add=True)

  def wait_scatter(j):
    pltpu.make_async_copy(
        rows_bufs[j], acc.at[didx_v.at[0]], ssem).wait()

  for k in range(n_steps):
    pltpu.sync_copy(zeros_hbm, acc.at[pl.ds(base, rows_pt)])
    pltpu.sync_copy(sidx_hbm.at[k, wid], sidx_v)
    plsc.subcore_barrier()

    for j in range(NB):
      gather(j, j)

    def chunkgrp(i, carry):
      for j in range(NB):
        wait_gather(j)
        scatter(NB * i + j, j)
      for j in range(NB):
        wait_scatter(j)
        gather(lax.rem(NB * (i + 1) + j, nch), j)
      return carry
    lax.fori_loop(0, nch // NB - 1, chunkgrp, 0)
    for j in range(NB):
      wait_gather(j)
      scatter(nch - NB + j, j)
    for j in range(NB):
      wait_scatter(j)
    plsc.subcore_barrier()

    pltpu.sync_copy(acc.at[pl.ds(base, rows_pt)],
                    out_hbm.at[k * NC + c, pl.ds(base, rows_pt)])


def _spmm(tab, sidx, didx, acc_rows):
  n_steps = sidx.shape[0]
  nch = sidx.shape[2]
  C = tab.shape[1]
  mesh = plsc.VectorSubcoreMesh(
      core_axis_name="c", subcore_axis_name="s", num_cores=NC,
      num_subcores=NS)
  zeros = jnp.zeros((acc_rows // NS, C), F32)
  return pl.kernel(
      functools.partial(_spmm_body, n_steps, C),
      out_type=jax.ShapeDtypeStruct((n_steps * NC, acc_rows, C), F32),
      mesh=mesh,
      compiler_params=_SC_PARAMS,
      scratch_types=[
          pltpu.VMEM((nch, CHK), jnp.int32),
          pltpu.VMEM((nch, CHK), jnp.int32),
          pltpu.VMEM((CHK, C), F32),
          pltpu.VMEM((CHK, C), F32),
          pltpu.VMEM((CHK, C), F32),
          pltpu.VMEM((CHK, C), F32),
          pltpu.VMEM_SHARED((acc_rows, C), F32),
          pltpu.SemaphoreType.DMA,
          pltpu.SemaphoreType.DMA,
      ],
  )(tab, sidx, didx, zeros)


def _deg_body(C, ones_hbm, didx_hbm, out_hbm, didx_v, ones_v, zbuf_v, acc):
  c = lax.axis_index("c")
  s = lax.axis_index("s")
  wid = s * NC + c
  nch = didx_hbm.shape[1]
  rows_pt = acc.shape[0] // NS
  stage = rows_pt // 2
  base = s * rows_pt

  def zrow(i, carry):
    for j in range(C // 16):
      zbuf_v[i, pl.ds(j * 16, 16)] = jnp.zeros((16,), F32)
    return carry
  lax.fori_loop(0, stage, zrow, 0)

  pltpu.sync_copy(zbuf_v, acc.at[pl.ds(base, stage)])
  pltpu.sync_copy(zbuf_v, acc.at[pl.ds(base + stage, stage)])
  pltpu.sync_copy(ones_hbm, ones_v)
  pltpu.sync_copy(didx_hbm.at[wid], didx_v)
  plsc.subcore_barrier()

  def chunk(ch, carry):
    pltpu.sync_copy(ones_v, acc.at[didx_v.at[ch]], add=True)
    return carry
  lax.fori_loop(0, nch, chunk, 0)
  plsc.subcore_barrier()

  pltpu.sync_copy(acc.at[pl.ds(base, stage)], zbuf_v)
  pltpu.sync_copy(zbuf_v, out_hbm.at[c, pl.ds(base, stage)])
  pltpu.sync_copy(acc.at[pl.ds(base + stage, stage)], zbuf_v)
  pltpu.sync_copy(zbuf_v, out_hbm.at[c, pl.ds(base + stage, stage)])


def _deg(didx, acc_rows, C):
  nch = didx.shape[1]
  mesh = plsc.VectorSubcoreMesh(
      core_axis_name="c", subcore_axis_name="s", num_cores=NC,
      num_subcores=NS)
  ones = jnp.ones((CHK, C), F32)
  return pl.kernel(
      functools.partial(_deg_body, C),
      out_type=jax.ShapeDtypeStruct((NC, acc_rows, C), F32),
      mesh=mesh,
      compiler_params=_SC_PARAMS,
      scratch_types=[
          pltpu.VMEM((nch, CHK), jnp.int32),
          pltpu.VMEM((CHK, C), F32),
          pltpu.VMEM((acc_rows // NS // 2, C), F32),
          pltpu.VMEM_SHARED((acc_rows, C), F32),
      ],
  )(ones, didx)


def _dinv_body(dega_ref, degb_ref, out_ref):
  deg = dega_ref[0][:, 0:1] + degb_ref[0][:, 0:1]
  dinv = jnp.where(deg > 0, lax.rsqrt(deg), 0.0)
  out_ref[...] = jnp.broadcast_to(dinv, out_ref.shape)


def _xside_body(H, x_ref, w0_ref, dinv_ref, xw0_ref, xlo_ref, xhi_ref):
  x = x_ref[...]
  xw0_ref[...] = jnp.dot(x, w0_ref[...], preferred_element_type=F32)
  xs = x * dinv_ref[...]
  xlo_ref[...] = xs[:, :H]
  xhi_ref[...] = xs[:, H:]


def _sx_body(lla_ref, llb_ref, lha_ref, lhb_ref, dinv_ref, w1_ref, out_ref):
  lx = jnp.concatenate(
      [lla_ref[0] + llb_ref[0], lha_ref[0] + lhb_ref[0]], axis=1)
  lx = lx * dinv_ref[...]
  out_ref[...] = jnp.dot(lx, w1_ref[...], preferred_element_type=F32)


def _gate_body(H, sx_ref, xw0_ref, lha_ref, lhb_ref, dinv_ref, h_ref,
               bz_ref, br_ref, wh0zr_ref, whz1_ref, whr1_ref, w0_ref, w1_ref,
               z_ref, cw0_ref, uc1_ref):
  dinv = dinv_ref[...][:, :H]
  lh = (lha_ref[0] + lhb_ref[0]) * dinv
  shz = jnp.dot(lh, whz1_ref[...], preferred_element_type=F32)
  shr = jnp.dot(lh, whr1_ref[...], preferred_element_type=F32)
  h = h_ref[...]
  hw0 = jnp.dot(h, wh0zr_ref[...], preferred_element_type=F32)
  xw0 = xw0_ref[...]
  sx = sx_ref[...]
  z = jax.nn.sigmoid(
      xw0[:, :H] - sx[:, :H] + hw0[:, :H] - shz + bz_ref[...])
  r = jax.nn.sigmoid(
      xw0[:, H:2 * H] - sx[:, H:2 * H] + hw0[:, H:] - shr + br_ref[...])
  c = h * r
  z_ref[...] = z
  cw0_ref[...] = jnp.dot(c, w0_ref[...], preferred_element_type=F32)
  cs = c * dinv
  uc1_ref[...] = jnp.dot(cs, w1_ref[...], preferred_element_type=F32)


def _update_body(H, sx_ref, xw0_ref, cw0_ref, sca_ref, scb_ref,
                 dinv_ref, z_ref, h_ref, bh_ref, out_ref, hs_ref):
  dinv = dinv_ref[...][:, :H]
  sc = (sca_ref[0] + scb_ref[0]) * dinv
  ht = jnp.tanh(xw0_ref[...][:, 2 * H:] - sx_ref[...][:, 2 * H:]
                + cw0_ref[...] - sc + bh_ref[...])
  z = z_ref[...]
  hn = z * h_ref[...] + (1.0 - z) * ht
  out_ref[...] = hn
  hs_ref[...] = hn * dinv


def _step0_body(H, sx_ref, xw0_ref, dinv_ref, bz_ref, bh_ref, out_ref,
                hs_ref):
  z = jax.nn.sigmoid(xw0_ref[...][:, :H] - sx_ref[...][:, :H] + bz_ref[...])
  ht = jnp.tanh(xw0_ref[...][:, 2 * H:] - sx_ref[...][:, 2 * H:]
                + bh_ref[...])
  hn = (1.0 - z) * ht
  out_ref[...] = hn
  hs_ref[...] = hn * dinv_ref[...][:, :H]


def _final_body(h_ref, wl_ref, bl_ref, out_ref):
  h = jnp.maximum(h_ref[...], 0.0)
  out_ref[...] = (jnp.dot(h, wl_ref[...], preferred_element_type=F32)
                  + bl_ref[...])


def kernel(x_seq, Wxz, bxz, Whz, bhz, Wxr, bxr, Whr, bhr, Wxh, bxh, Whh, bhh,
           Wl, bl, edge_index):
  if x_seq.ndim == 3:
    x_seq = x_seq[None]
  B, T, n, F = x_seq.shape
  H = Whz.shape[2]
  OUT = Wl.shape[1]
  BT = B * T
  nblk = n // BM
  acc_rows = ((n + 1 + 16 * NS - 1) // (16 * NS)) * (16 * NS)

  E = edge_index.shape[1]
  nch = -(-E // (NW * CHK))
  nch += (-nch) % 4
  Epad = NW * CHK * nch
  src = edge_index[0].astype(jnp.int32)
  dst = edge_index[1].astype(jnp.int32)
  pad0 = jnp.zeros((Epad - E,), jnp.int32)
  padn = jnp.full((Epad - E,), n, jnp.int32)
  sidx = jnp.concatenate([src, pad0]).reshape(NW, nch, CHK)
  didx = jnp.concatenate([dst, padn]).reshape(NW, nch, CHK)
  sdeg = jnp.concatenate([src, padn]).reshape(NW, nch, CHK)
  offs_bt = (jnp.arange(BT, dtype=jnp.int32) * n)[:, None, None, None]
  sidx_bt = sidx[None] + offs_bt
  sidx_1 = sidx[None]

  deg2 = _deg(sdeg, acc_rows, 16)
  dinvmat = pl.pallas_call(
      _dinv_body,
      grid=(nblk,),
      in_specs=[
          pl.BlockSpec((1, BM, 16), lambda i: (0, i, 0)),
          pl.BlockSpec((1, BM, 16), lambda i: (1, i, 0)),
      ],
      out_specs=pl.BlockSpec((BM, 128), lambda i: (i, 0)),
      out_shape=jax.ShapeDtypeStruct((n, 128), F32),
  )(deg2, deg2)

  X2 = x_seq.reshape(BT * n, F)
  W0cat = jnp.concatenate([Wxz[0], Wxr[0], Wxh[0]], axis=1)
  W1cat = jnp.concatenate([Wxz[1], Wxr[1], Wxh[1]], axis=1)
  XW0, XsLo, XsHi = pl.pallas_call(
      functools.partial(_xside_body, H),
      grid=(BT * nblk,),
      in_specs=[
          pl.BlockSpec((BM, F), lambda i: (i, 0)),
          pl.BlockSpec((F, 3 * H), lambda i: (0, 0)),
          pl.BlockSpec((BM, 128), lambda i: (i % nblk, 0)),
      ],
      out_specs=[
          pl.BlockSpec((BM, 3 * H), lambda i: (i, 0)),
          pl.BlockSpec((BM, H), lambda i: (i, 0)),
          pl.BlockSpec((BM, H), lambda i: (i, 0)),
      ],
      out_shape=[
          jax.ShapeDtypeStruct((BT * n, 3 * H), F32),
          jax.ShapeDtypeStruct((BT * n, H), F32),
          jax.ShapeDtypeStruct((BT * n, H), F32),
      ],
  )(X2, W0cat, dinvmat)

  LxLo = _spmm(XsLo, sidx_bt, didx, acc_rows)
  LxHi = _spmm(XsHi, sidx_bt, didx, acc_rows)

  SX = pl.pallas_call(
      _sx_body,
      grid=(BT * nblk,),
      in_specs=[
          pl.BlockSpec((1, BM, H), lambda i: (2 * (i // nblk), i % nblk, 0)),
          pl.BlockSpec((1, BM, H),
                       lambda i: (2 * (i // nblk) + 1, i % nblk, 0)),
          pl.BlockSpec((1, BM, H), lambda i: (2 * (i // nblk), i % nblk, 0)),
          pl.BlockSpec((1, BM, H),
                       lambda i: (2 * (i // nblk) + 1, i % nblk, 0)),
          pl.BlockSpec((BM, 128), lambda i: (i % nblk, 0)),
          pl.BlockSpec((F, 3 * H), lambda i: (0, 0)),
      ],
      out_specs=pl.BlockSpec((BM, 3 * H), lambda i: (i, 0)),
      out_shape=jax.ShapeDtypeStruct((BT * n, 3 * H), F32),
  )(LxLo, LxLo, LxHi, LxHi, dinvmat, W1cat)

  Wh0zr = jnp.concatenate([Whz[0], Whr[0]], axis=1)
  bz2 = (bxz + bhz).reshape(1, H)
  br2 = (bxr + bhr).reshape(1, H)
  bh2 = (bxh + bhh).reshape(1, H)
  bl2 = bl.reshape(1, OUT)

  outs = []
  for b in range(B):
    bt0 = b * T
    h, hs = pl.pallas_call(
        functools.partial(_step0_body, H),
        grid=(nblk,),
        in_specs=[
            pl.BlockSpec((BM, 3 * H), lambda i, bt=bt0: (bt * nblk + i, 0)),
            pl.BlockSpec((BM, 3 * H), lambda i, bt=bt0: (bt * nblk + i, 0)),
            pl.BlockSpec((BM, 128), lambda i: (i, 0)),
            pl.BlockSpec((1, H), lambda i: (0, 0)),
            pl.BlockSpec((1, H), lambda i: (0, 0)),
        ],
        out_specs=[
            pl.BlockSpec((BM, H), lambda i: (i, 0)),
            pl.BlockSpec((BM, H), lambda i: (i, 0)),
        ],
        out_shape=[
            jax.ShapeDtypeStruct((n, H), F32),
            jax.ShapeDtypeStruct((n, H), F32),
        ],
    )(SX, XW0, dinvmat, bz2, bh2)
    for t in range(1, T):
      bt = b * T + t
      Lh = _spmm(hs, sidx_1, didx, acc_rows)

      Z, CW0, UC1 = pl.pallas_call(
          functools.partial(_gate_body, H),
          grid=(nblk,),
          in_specs=[
              pl.BlockSpec((BM, 3 * H), lambda i, bt=bt: (bt * nblk + i, 0)),
              pl.BlockSpec((BM, 3 * H), lambda i, bt=bt: (bt * nblk + i, 0)),
              pl.BlockSpec((1, BM, H), lambda i: (0, i, 0)),
              pl.BlockSpec((1, BM, H), lambda i: (1, i, 0)),
              pl.BlockSpec((BM, 128), lambda i: (i, 0)),
              pl.BlockSpec((BM, H), lambda i: (i, 0)),
              pl.BlockSpec((1, H), lambda i: (0, 0)),
              pl.BlockSpec((1, H), lambda i: (0, 0)),
              pl.BlockSpec((H, 2 * H), lambda i: (0, 0)),
              pl.BlockSpec((H, H), lambda i: (0, 0)),
              pl.BlockSpec((H, H), lambda i: (0, 0)),
              pl.BlockSpec((H, H), lambda i: (0, 0)),
              pl.BlockSpec((H, H), lambda i: (0, 0)),
          ],
          out_specs=[
              pl.BlockSpec((BM, H), lambda i: (i, 0)),
              pl.BlockSpec((BM, H), lambda i: (i, 0)),
              pl.BlockSpec((BM, H), lambda i: (i, 0)),
          ],
          out_shape=[
              jax.ShapeDtypeStruct((n, H), F32),
              jax.ShapeDtypeStruct((n, H), F32),
              jax.ShapeDtypeStruct((n, H), F32),
          ],
      )(SX, XW0, Lh, Lh, dinvmat, h, bz2, br2, Wh0zr, Whz[1], Whr[1],
        Whh[0], Whh[1])

      SCt = _spmm(UC1, sidx_1, didx, acc_rows)

      h, hs = pl.pallas_call(
          functools.partial(_update_body, H),
          grid=(nblk,),
          in_specs=[
              pl.BlockSpec((BM, 3 * H), lambda i, bt=bt: (bt * nblk + i, 0)),
              pl.BlockSpec((BM, 3 * H), lambda i, bt=bt: (bt * nblk + i, 0)),
              pl.BlockSpec((BM, H), lambda i: (i, 0)),
              pl.BlockSpec((1, BM, H), lambda i: (0, i, 0)),
              pl.BlockSpec((1, BM, H), lambda i: (1, i, 0)),
              pl.BlockSpec((BM, 128), lambda i: (i, 0)),
              pl.BlockSpec((BM, H), lambda i: (i, 0)),
              pl.BlockSpec((BM, H), lambda i: (i, 0)),
              pl.BlockSpec((1, H), lambda i: (0, 0)),
          ],
          out_specs=[
              pl.BlockSpec((BM, H), lambda i: (i, 0)),
              pl.BlockSpec((BM, H), lambda i: (i, 0)),
          ],
          out_shape=[
              jax.ShapeDtypeStruct((n, H), F32),
              jax.ShapeDtypeStruct((n, H), F32),
          ],
      )(SX, XW0, CW0, SCt, SCt, dinvmat, Z, h, bh2)

    out_b = pl.pallas_call(
        _final_body,
        grid=(nblk,),
        in_specs=[
            pl.BlockSpec((BM, H), lambda i: (i, 0)),
            pl.BlockSpec((H, OUT), lambda i: (0, 0)),
            pl.BlockSpec((1, OUT), lambda i: (0, 0)),
        ],
        out_specs=pl.BlockSpec((BM, OUT), lambda i: (i, 0)),
        out_shape=jax.ShapeDtypeStruct((n, OUT), F32),
    )(h, Wl, bl2)
    outs.append(out_b)

  return jnp.stack(outs, axis=0)
